# Initial kernel scaffold; baseline (speedup 1.0000x reference)
import jax
import jax.numpy as jnp
from jax.experimental import pallas as pl


def kernel(x, edge_index, batch, W1, b1, gamma1, beta1, W2, b2, gamma2, beta2):
    raise NotImplementedError("write your pallas kernel here")


# speedup vs baseline: 6.0915x; 6.0915x over previous
import functools

import jax
import jax.numpy as jnp
from jax import lax
from jax.experimental import pallas as pl
from jax.experimental.pallas import tpu as pltpu
from jax.experimental.pallas import tpu_sc as plsc

N = 10000
NPAD = 10240
E = 160000
EPAD = 163840
IN_C = 256
HID = 1024
OUT = 128
NG = 16
DUMP = N

NC, NS = 2, 16
CH = 128
ROWS_PER_TILE = NPAD // (NC * NS)

_SC_MESH = dict(mesh=plsc.VectorSubcoreMesh(core_axis_name="c", subcore_axis_name="s"))

SLAB = NPAD // NS


@functools.partial(
    pl.kernel,
    out_type=jax.ShapeDtypeStruct((NC, NPAD), jnp.float32),
    scratch_types=[
        pltpu.VMEM_SHARED((NPAD,), jnp.float32),
        pltpu.VMEM((SLAB,), jnp.float32),
        pltpu.VMEM((CH,), jnp.float32),
        pltpu.VMEM((CH,), jnp.int32),
    ],
    **_SC_MESH,
)
def _deg_sc(dst_hbm, out_hbm, acc, zbuf, onesv, idx_d):
    c = lax.axis_index("c")
    s = lax.axis_index("s")
    for k in range(SLAB // 16):
        zbuf[pl.ds(k * 16, 16)] = jnp.zeros((16,), jnp.float32)
    for k in range(CH // 16):
        onesv[pl.ds(k * 16, 16)] = jnp.ones((16,), jnp.float32)
    pltpu.sync_copy(zbuf, acc.at[pl.ds(s * SLAB, SLAB)])
    plsc.subcore_barrier()
    per_tile = EPAD // (NC * NS)
    base = (c * NS + s) * per_tile

    def body(j, carry):
        off = base + j * CH
        pltpu.sync_copy(dst_hbm.at[pl.ds(off, CH)], idx_d)
        pltpu.sync_copy(onesv, acc.at[idx_d], add=True)
        return carry

    lax.fori_loop(0, per_tile // CH, body, 0)
    plsc.subcore_barrier()
    pltpu.sync_copy(acc.at[pl.ds(s * SLAB, SLAB)], out_hbm.at[c, pl.ds(s * SLAB, SLAB)])


def _prep_body(deg0_ref, deg1_ref, x_ref, xsa_ref, xsb_ref, dinv_ref):
    deg = deg0_ref[...] + deg1_ref[...] + 1.0
    dinv = lax.rsqrt(deg)
    dinv_ref[...] = dinv
    xs = x_ref[...] * dinv
    xsa_ref[...] = xs[:, :OUT]
    xsb_ref[...] = xs[:, OUT:]


_BLK = 512
_NBLK = NPAD // _BLK


def _prep_tc(deg0, deg1, x_pad):
    return pl.pallas_call(
        _prep_body,
        grid=(_NBLK,),
        in_specs=[
            pl.BlockSpec((_BLK, 1), lambda i: (i, 0)),
            pl.BlockSpec((_BLK, 1), lambda i: (i, 0)),
            pl.BlockSpec((_BLK, IN_C), lambda i: (i, 0)),
        ],
        out_specs=[
            pl.BlockSpec((_BLK, OUT), lambda i: (i, 0)),
            pl.BlockSpec((_BLK, OUT), lambda i: (i, 0)),
            pl.BlockSpec((_BLK, 1), lambda i: (i, 0)),
        ],
        out_shape=[
            jax.ShapeDtypeStruct((NPAD, OUT), jnp.float32),
            jax.ShapeDtypeStruct((NPAD, OUT), jnp.float32),
            jax.ShapeDtypeStruct((NPAD, 1), jnp.float32),
        ],
    )(deg0, deg1, x_pad)


@functools.partial(
    pl.kernel,
    out_type=jax.ShapeDtypeStruct((NC, NPAD, OUT), jnp.float32),
    scratch_types=[
        pltpu.VMEM_SHARED((NPAD, OUT), jnp.float32),
        pltpu.VMEM((CH,), jnp.int32),
        pltpu.VMEM((CH,), jnp.int32),
        pltpu.VMEM((CH, OUT), jnp.float32),
        pltpu.SemaphoreType.DMA,
    ],
    **_SC_MESH,
)
def _agg1_sc(src_hbm, dst_hbm, xsa_hbm, xsb_hbm, out_hbm, acc, idx_s, idx_d, rows, sem):
    c = lax.axis_index("c")
    s = lax.axis_index("s")

    @pl.when(c == 0)
    def _():
        pltpu.sync_copy(xsa_hbm.at[pl.ds(s * SLAB, SLAB)], acc.at[pl.ds(s * SLAB, SLAB)])

    @pl.when(c == 1)
    def _():
        pltpu.sync_copy(xsb_hbm.at[pl.ds(s * SLAB, SLAB)], acc.at[pl.ds(s * SLAB, SLAB)])

    plsc.subcore_barrier()

    per_tile = EPAD // NS
    base = s * per_tile

    def body(j, carry):
        off = base + j * CH
        pltpu.sync_copy(src_hbm.at[pl.ds(off, CH)], idx_s)
        pltpu.sync_copy(dst_hbm.at[pl.ds(off, CH)], idx_d)

        @pl.when(c == 0)
        def _():
            pltpu.async_copy(xsa_hbm.at[idx_s], rows, sem).wait()

        @pl.when(c == 1)
        def _():
            pltpu.async_copy(xsb_hbm.at[idx_s], rows, sem).wait()

        pltpu.sync_copy(rows, acc.at[idx_d], add=True)
        return carry

    lax.fori_loop(0, per_tile // CH, body, 0)
    plsc.subcore_barrier()
    pltpu.sync_copy(acc.at[pl.ds(s * SLAB, SLAB)], out_hbm.at[c, pl.ds(s * SLAB, SLAB)])


@functools.partial(
    pl.kernel,
    out_type=jax.ShapeDtypeStruct((NC, NPAD, OUT), jnp.float32),
    scratch_types=[
        pltpu.VMEM_SHARED((NPAD, OUT), jnp.float32),
        pltpu.VMEM((CH,), jnp.int32),
        pltpu.VMEM((CH,), jnp.int32),
        pltpu.VMEM((CH, OUT), jnp.float32),
        pltpu.SemaphoreType.DMA,
    ],
    **_SC_MESH,
)
def _agg2_sc(src_hbm, dst_hbm, hws_hbm, out_hbm, acc, idx_s, idx_d, rows, sem):
    c = lax.axis_index("c")
    s = lax.axis_index("s")

    pltpu.sync_copy(hws_hbm.at[pl.ds(s * SLAB, SLAB)], acc.at[pl.ds(s * SLAB, SLAB)])
    plsc.subcore_barrier()

    per_tile = EPAD // (NC * NS)
    base = (c * NS + s) * per_tile

    def body(j, carry):
        off = base + j * CH
        pltpu.sync_copy(src_hbm.at[pl.ds(off, CH)], idx_s)
        pltpu.sync_copy(dst_hbm.at[pl.ds(off, CH)], idx_d)
        pltpu.async_copy(hws_hbm.at[idx_s], rows, sem).wait()
        pltpu.sync_copy(rows, acc.at[idx_d], add=True)
        return carry

    lax.fori_loop(0, per_tile // CH, body, 0)
    plsc.subcore_barrier()
    pltpu.sync_copy(acc.at[pl.ds(s * SLAB, SLAB)], out_hbm.at[c, pl.ds(s * SLAB, SLAB)])


def _layer1_body(pa_ref, pb_ref, dinv_ref, w1_ref, b1_ref, h_ref, stats_ref, ssum, ssq):
    i = pl.program_id(0)
    dv = dinv_ref[...]
    pa = pa_ref[...] * dv
    pb = pb_ref[...] * dv
    w1 = w1_ref[...]
    h = (
        jnp.dot(pa, w1[:OUT, :], preferred_element_type=jnp.float32)
        + jnp.dot(pb, w1[OUT:, :], preferred_element_type=jnp.float32)
        + b1_ref[...]
    )
    h_ref[...] = h
    rows = i * _BLK + lax.broadcasted_iota(jnp.int32, (_BLK, 1), 0)
    hm = jnp.where(rows < N, h, 0.0)

    @pl.when(i == 0)
    def _():
        ssum[...] = jnp.zeros_like(ssum)
        ssq[...] = jnp.zeros_like(ssq)

    ssum[...] += jnp.sum(hm, axis=0, keepdims=True)
    ssq[...] += jnp.sum(hm * hm, axis=0, keepdims=True)

    @pl.when(i == _NBLK - 1)
    def _():
        stats_ref[...] = jnp.concatenate([ssum[...], ssq[...]], axis=0)


def _layer1_tc(pa, pb, dinv, W1, b1):
    return pl.pallas_call(
        _layer1_body,
        grid=(_NBLK,),
        in_specs=[
            pl.BlockSpec((_BLK, OUT), lambda i: (i, 0)),
            pl.BlockSpec((_BLK, OUT), lambda i: (i, 0)),
            pl.BlockSpec((_BLK, 1), lambda i: (i, 0)),
            pl.BlockSpec((IN_C, HID), lambda i: (0, 0)),
            pl.BlockSpec((1, HID), lambda i: (0, 0)),
        ],
        out_specs=[
            pl.BlockSpec((_BLK, HID), lambda i: (i, 0)),
            pl.BlockSpec((2, HID), lambda i: (0, 0)),
        ],
        out_shape=[
            jax.ShapeDtypeStruct((NPAD, HID), jnp.float32),
            jax.ShapeDtypeStruct((2, HID), jnp.float32),
        ],
        scratch_shapes=[
            pltpu.VMEM((1, HID), jnp.float32),
            pltpu.VMEM((1, HID), jnp.float32),
        ],
    )(pa, pb, dinv, W1, b1)


def _layer2a_body(h_ref, stats_ref, dinv_ref, w2_ref, g1_ref, be1_ref, hws_ref):
    stats = stats_ref[...]
    mu = stats[0:1, :] * (1.0 / N)
    var = stats[1:2, :] * (1.0 / N) - mu * mu
    alpha = g1_ref[...] * lax.rsqrt(var + 1e-5)
    c0 = be1_ref[...] - mu * alpha
    hn = jnp.maximum(h_ref[...] * alpha + c0, 0.0)
    hw = jnp.dot(hn, w2_ref[...], preferred_element_type=jnp.float32)
    hws_ref[...] = hw * dinv_ref[...]


def _layer2a_tc(h, stats, dinv, W2, g1, be1):
    return pl.pallas_call(
        _layer2a_body,
        grid=(_NBLK,),
        in_specs=[
            pl.BlockSpec((_BLK, HID), lambda i: (i, 0)),
            pl.BlockSpec((2, HID), lambda i: (0, 0)),
            pl.BlockSpec((_BLK, 1), lambda i: (i, 0)),
            pl.BlockSpec((HID, OUT), lambda i: (0, 0)),
            pl.BlockSpec((1, HID), lambda i: (0, 0)),
            pl.BlockSpec((1, HID), lambda i: (0, 0)),
        ],
        out_specs=pl.BlockSpec((_BLK, OUT), lambda i: (i, 0)),
        out_shape=jax.ShapeDtypeStruct((NPAD, OUT), jnp.float32),
    )(h, stats, dinv, W2, g1, be1)


def _final_body(p20_ref, p21_ref, hws_ref, dinv_ref, batch_ref, g2_ref, be2_ref,
                out_ref, ssum, ssq, pooled, cntf):
    i = pl.program_id(0)
    a = (p20_ref[...] + p21_ref[...] - hws_ref[...]) * dinv_ref[...]
    rows = i * _BLK + lax.broadcasted_iota(jnp.int32, (_BLK, 1), 0)
    mask = rows < N
    am = jnp.where(mask, a, 0.0)

    @pl.when(i == 0)
    def _():
        ssum[...] = jnp.zeros_like(ssum)
        ssq[...] = jnp.zeros_like(ssq)
        pooled[...] = jnp.zeros_like(pooled)
        cntf[...] = jnp.zeros_like(cntf)

    ssum[...] += jnp.sum(am, axis=0, keepdims=True)
    ssq[...] += jnp.sum(am * am, axis=0, keepdims=True)
    gids = lax.broadcasted_iota(jnp.int32, (_BLK, NG), 1)
    onehot = jnp.where((batch_ref[...] == gids) & mask, 1.0, 0.0)
    pooled[...] += lax.dot_general(onehot, am, (((0,), (0,)), ((), ())),
                                   preferred_element_type=jnp.float32)
    maskb = jnp.where(mask, 1.0, 0.0) * jnp.ones((_BLK, OUT), jnp.float32)
    cntf[...] += lax.dot_general(onehot, maskb, (((0,), (0,)), ((), ())),
                                 preferred_element_type=jnp.float32)

    @pl.when(i == _NBLK - 1)
    def _():
        mu = ssum[...] * (1.0 / N)
        var = ssq[...] * (1.0 / N) - mu * mu
        inv = lax.rsqrt(var + 1e-5)
        cnt = cntf[...]
        pm = pooled[...] / jnp.maximum(cnt, 1.0)
        res = (pm - mu) * inv * g2_ref[...] + be2_ref[...]
        out_ref[...] = jnp.where(cnt > 0.0, res, 0.0)


def _final_tc(p20, p21, hws, dinv, batchp, g2, be2):
    return pl.pallas_call(
        _final_body,
        grid=(_NBLK,),
        in_specs=[
            pl.BlockSpec((_BLK, OUT), lambda i: (i, 0)),
            pl.BlockSpec((_BLK, OUT), lambda i: (i, 0)),
            pl.BlockSpec((_BLK, OUT), lambda i: (i, 0)),
            pl.BlockSpec((_BLK, 1), lambda i: (i, 0)),
            pl.BlockSpec((_BLK, 1), lambda i: (i, 0)),
            pl.BlockSpec((1, OUT), lambda i: (0, 0)),
            pl.BlockSpec((1, OUT), lambda i: (0, 0)),
        ],
        out_specs=pl.BlockSpec((NG, OUT), lambda i: (0, 0)),
        out_shape=jax.ShapeDtypeStruct((NG, OUT), jnp.float32),
        scratch_shapes=[
            pltpu.VMEM((1, OUT), jnp.float32),
            pltpu.VMEM((1, OUT), jnp.float32),
            pltpu.VMEM((NG, OUT), jnp.float32),
            pltpu.VMEM((NG, OUT), jnp.float32),
        ],
    )(p20, p21, hws, dinv, batchp, g2, be2)


def kernel(x, edge_index, batch, W1, b1, gamma1, beta1, W2, b2, gamma2, beta2):
    del b2
    src = edge_index[0]
    dst = edge_index[1]
    pad_e = jnp.full((EPAD - E,), DUMP, jnp.int32)
    srcp = jnp.concatenate([src, pad_e])
    dstp = jnp.concatenate([dst, pad_e])
    x_pad = jnp.pad(x, ((0, NPAD - N), (0, 0)))
    batchp = jnp.pad(batch, (0, NPAD - N)).reshape(NPAD, 1)

    degp = _deg_sc(dstp)
    deg0 = degp[0].reshape(NPAD, 1)
    deg1 = degp[1].reshape(NPAD, 1)
    xsa, xsb, dinv = _prep_tc(deg0, deg1, x_pad)
    p1 = _agg1_sc(srcp, dstp, xsa, xsb)
    h, stats1 = _layer1_tc(p1[0], p1[1], dinv, W1, b1.reshape(1, HID))
    hws = _layer2a_tc(h, stats1, dinv, W2,
                      gamma1.reshape(1, HID), beta1.reshape(1, HID))
    p2 = _agg2_sc(srcp, dstp, hws)
    out = _final_tc(p2[0], p2[1], hws, dinv, batchp,
                    gamma2.reshape(1, OUT), beta2.reshape(1, OUT))
    return out


# speedup vs baseline: 7.0823x; 1.1627x over previous
import functools

import jax
import jax.numpy as jnp
from jax import lax
from jax.experimental import pallas as pl
from jax.experimental.pallas import tpu as pltpu
from jax.experimental.pallas import tpu_sc as plsc

N = 10000
NPAD = 10240
E = 160000
EPAD = 163840
IN_C = 256
HID = 1024
OUT = 128
NG = 16
DUMP = N

NC, NS = 2, 16
CH = 64
NBUF = 4
GRP = 40
ECH = EPAD // CH
SLAB = NPAD // NS

_SC_MESH = dict(mesh=plsc.VectorSubcoreMesh(core_axis_name="c", subcore_axis_name="s"))


@functools.partial(
    pl.kernel,
    out_type=jax.ShapeDtypeStruct((NC, NPAD), jnp.float32),
    scratch_types=[
        pltpu.VMEM_SHARED((NPAD,), jnp.float32),
        pltpu.VMEM((SLAB,), jnp.float32),
        pltpu.VMEM((CH,), jnp.float32),
        pltpu.VMEM((EPAD // (NC * NS) // CH, CH), jnp.int32),
    ]
    + [pltpu.SemaphoreType.DMA] * NBUF,
    **_SC_MESH,
)
def _deg_sc(dst_hbm, out_hbm, acc, zbuf, onesv, idxd, s0, s1, s2, s3):
    c = lax.axis_index("c")
    s = lax.axis_index("s")
    ssems = (s0, s1, s2, s3)
    per_tile = EPAD // (NC * NS)
    nch = per_tile // CH
    for k in range(SLAB // 16):
        zbuf[pl.ds(k * 16, 16)] = jnp.zeros((16,), jnp.float32)
    for k in range(CH // 16):
        onesv[pl.ds(k * 16, 16)] = jnp.ones((16,), jnp.float32)
    tile = c * NS + s
    pltpu.sync_copy(dst_hbm.at[pl.ds(tile * (nch), nch)], idxd)
    pltpu.sync_copy(zbuf, acc.at[pl.ds(s * SLAB, SLAB)])
    plsc.subcore_barrier()

    def _wait_sc(b):
        pltpu.make_async_copy(onesv, acc.at[idxd.at[0]], ssems[b]).wait()

    for b in range(NBUF):
        pltpu.async_copy(onesv, acc.at[idxd.at[b]], ssems[b], add=True)

    @pl.loop(0, nch // NBUF - 1)
    def _(r):
        for b in range(NBUF):
            _wait_sc(b)
            pltpu.async_copy(onesv, acc.at[idxd.at[(r + 1) * NBUF + b]], ssems[b], add=True)

    for b in range(NBUF):
        _wait_sc(b)
    plsc.subcore_barrier()
    pltpu.sync_copy(acc.at[pl.ds(s * SLAB, SLAB)], out_hbm.at[c, pl.ds(s * SLAB, SLAB)])


def _agg_pipeline(table_hbm, src_hbm, dst_hbm, chunk0, nch,
                  idxs, idxd, rows, gsems, ssems, acc):
    def _wait_g(b):
        pltpu.make_async_copy(table_hbm.at[idxs.at[0]], rows[b], gsems[b]).wait()

    def _wait_s(b):
        pltpu.make_async_copy(rows[b], acc.at[idxd.at[0]], ssems[b]).wait()

    @pl.loop(0, nch // GRP)
    def _(g):
        pltpu.sync_copy(src_hbm.at[pl.ds(chunk0 + g * GRP, GRP)], idxs)
        pltpu.sync_copy(dst_hbm.at[pl.ds(chunk0 + g * GRP, GRP)], idxd)

        for b in range(NBUF):
            pltpu.async_copy(table_hbm.at[idxs.at[b]], rows[b], gsems[b])

        @pl.loop(0, GRP // NBUF)
        def _(r):
            for b in range(NBUF):
                _wait_g(b)
                pltpu.async_copy(rows[b], acc.at[idxd.at[r * NBUF + b]], ssems[b], add=True)
            for b in range(NBUF):
                nxt = (r + 1) * NBUF + b
                _wait_s(b)

                @pl.when(nxt < GRP)
                def _():
                    pltpu.async_copy(table_hbm.at[idxs.at[nxt]], rows[b], gsems[b])


_C_NCH = EPAD // NS // CH


@functools.partial(
    pl.kernel,
    out_type=jax.ShapeDtypeStruct((NC, NPAD, OUT), jnp.float32),
    scratch_types=[
        pltpu.VMEM_SHARED((NPAD, OUT), jnp.float32),
        pltpu.VMEM((GRP, CH), jnp.int32),
        pltpu.VMEM((GRP, CH), jnp.int32),
    ]
    + [pltpu.VMEM((CH, OUT), jnp.float32)] * NBUF
    + [pltpu.SemaphoreType.DMA] * (2 * NBUF),
    **_SC_MESH,
)
def _agg1_sc(src_hbm, dst_hbm, xsa_hbm, xsb_hbm, out_hbm, acc, idxs, idxd,
             r0, r1, r2, r3, g0, g1, g2, g3, ss0, ss1, ss2, ss3):
    c = lax.axis_index("c")
    s = lax.axis_index("s")
    rows = (r0, r1, r2, r3)
    gsems = (g0, g1, g2, g3)
    ssems = (ss0, ss1, ss2, ss3)

    @pl.when(c == 0)
    def _():
        pltpu.sync_copy(xsa_hbm.at[pl.ds(s * SLAB, SLAB)], acc.at[pl.ds(s * SLAB, SLAB)])

    @pl.when(c == 1)
    def _():
        pltpu.sync_copy(xsb_hbm.at[pl.ds(s * SLAB, SLAB)], acc.at[pl.ds(s * SLAB, SLAB)])

    plsc.subcore_barrier()

    @pl.when(c == 0)
    def _():
        _agg_pipeline(xsa_hbm, src_hbm, dst_hbm, s * _C_NCH, _C_NCH,
                      idxs, idxd, rows, gsems, ssems, acc)

    @pl.when(c == 1)
    def _():
        _agg_pipeline(xsb_hbm, src_hbm, dst_hbm, s * _C_NCH, _C_NCH,
                      idxs, idxd, rows, gsems, ssems, acc)

    plsc.subcore_barrier()
    pltpu.sync_copy(acc.at[pl.ds(s * SLAB, SLAB)], out_hbm.at[c, pl.ds(s * SLAB, SLAB)])


_F_NCH = EPAD // (NC * NS) // CH


@functools.partial(
    pl.kernel,
    out_type=jax.ShapeDtypeStruct((NC, NPAD, OUT), jnp.float32),
    scratch_types=[
        pltpu.VMEM_SHARED((NPAD, OUT), jnp.float32),
        pltpu.VMEM((GRP, CH), jnp.int32),
        pltpu.VMEM((GRP, CH), jnp.int32),
    ]
    + [pltpu.VMEM((CH, OUT), jnp.float32)] * NBUF
    + [pltpu.SemaphoreType.DMA] * (2 * NBUF),
    **_SC_MESH,
)
def _agg2_sc(src_hbm, dst_hbm, hws_hbm, out_hbm, acc, idxs, idxd,
             r0, r1, r2, r3, g0, g1, g2, g3, ss0, ss1, ss2, ss3):
    c = lax.axis_index("c")
    s = lax.axis_index("s")
    rows = (r0, r1, r2, r3)
    gsems = (g0, g1, g2, g3)
    ssems = (ss0, ss1, ss2, ss3)

    tile = c * NS + s
    pltpu.sync_copy(hws_hbm.at[pl.ds(s * SLAB, SLAB)], acc.at[pl.ds(s * SLAB, SLAB)])
    plsc.subcore_barrier()

    _agg_pipeline(hws_hbm, src_hbm, dst_hbm, tile * _F_NCH, _F_NCH,
                  idxs, idxd, rows, gsems, ssems, acc)

    plsc.subcore_barrier()
    pltpu.sync_copy(acc.at[pl.ds(s * SLAB, SLAB)], out_hbm.at[c, pl.ds(s * SLAB, SLAB)])


_BLK = 512
_NBLK = NPAD // _BLK


def _prep_body(deg0_ref, deg1_ref, x_ref, xsa_ref, xsb_ref, dinv_ref):
    deg = deg0_ref[...] + deg1_ref[...] + 1.0
    dinv = lax.rsqrt(deg)
    dinv_ref[...] = dinv
    xs = x_ref[...] * dinv
    xsa_ref[...] = xs[:, :OUT]
    xsb_ref[...] = xs[:, OUT:]


def _prep_tc(deg0, deg1, x_pad):
    return pl.pallas_call(
        _prep_body,
        grid=(_NBLK,),
        in_specs=[
            pl.BlockSpec((_BLK, 1), lambda i: (i, 0)),
            pl.BlockSpec((_BLK, 1), lambda i: (i, 0)),
            pl.BlockSpec((_BLK, IN_C), lambda i: (i, 0)),
        ],
        out_specs=[
            pl.BlockSpec((_BLK, OUT), lambda i: (i, 0)),
            pl.BlockSpec((_BLK, OUT), lambda i: (i, 0)),
            pl.BlockSpec((_BLK, 1), lambda i: (i, 0)),
        ],
        out_shape=[
            jax.ShapeDtypeStruct((NPAD, OUT), jnp.float32),
            jax.ShapeDtypeStruct((NPAD, OUT), jnp.float32),
            jax.ShapeDtypeStruct((NPAD, 1), jnp.float32),
        ],
    )(deg0, deg1, x_pad)


def _layer1_body(pa_ref, pb_ref, dinv_ref, w1_ref, b1_ref, h_ref, stats_ref, ssum, ssq):
    i = pl.program_id(0)
    dv = dinv_ref[...]
    pa = pa_ref[...] * dv
    pb = pb_ref[...] * dv
    w1 = w1_ref[...]
    h = (
        jnp.dot(pa, w1[:OUT, :], preferred_element_type=jnp.float32)
        + jnp.dot(pb, w1[OUT:, :], preferred_element_type=jnp.float32)
        + b1_ref[...]
    )
    h_ref[...] = h
    rows = i * _BLK + lax.broadcasted_iota(jnp.int32, (_BLK, 1), 0)
    hm = jnp.where(rows < N, h, 0.0)

    @pl.when(i == 0)
    def _():
        ssum[...] = jnp.zeros_like(ssum)
        ssq[...] = jnp.zeros_like(ssq)

    ssum[...] += jnp.sum(hm, axis=0, keepdims=True)
    ssq[...] += jnp.sum(hm * hm, axis=0, keepdims=True)

    @pl.when(i == _NBLK - 1)
    def _():
        stats_ref[...] = jnp.concatenate([ssum[...], ssq[...]], axis=0)


def _layer1_tc(pa, pb, dinv, W1, b1):
    return pl.pallas_call(
        _layer1_body,
        grid=(_NBLK,),
        in_specs=[
            pl.BlockSpec((_BLK, OUT), lambda i: (i, 0)),
            pl.BlockSpec((_BLK, OUT), lambda i: (i, 0)),
            pl.BlockSpec((_BLK, 1), lambda i: (i, 0)),
            pl.BlockSpec((IN_C, HID), lambda i: (0, 0)),
            pl.BlockSpec((1, HID), lambda i: (0, 0)),
        ],
        out_specs=[
            pl.BlockSpec((_BLK, HID), lambda i: (i, 0)),
            pl.BlockSpec((2, HID), lambda i: (0, 0)),
        ],
        out_shape=[
            jax.ShapeDtypeStruct((NPAD, HID), jnp.float32),
            jax.ShapeDtypeStruct((2, HID), jnp.float32),
        ],
        scratch_shapes=[
            pltpu.VMEM((1, HID), jnp.float32),
            pltpu.VMEM((1, HID), jnp.float32),
        ],
    )(pa, pb, dinv, W1, b1)


def _layer2a_body(h_ref, stats_ref, dinv_ref, w2_ref, g1_ref, be1_ref, hws_ref):
    stats = stats_ref[...]
    mu = stats[0:1, :] * (1.0 / N)
    var = stats[1:2, :] * (1.0 / N) - mu * mu
    alpha = g1_ref[...] * lax.rsqrt(var + 1e-5)
    c0 = be1_ref[...] - mu * alpha
    hn = jnp.maximum(h_ref[...] * alpha + c0, 0.0)
    hw = jnp.dot(hn, w2_ref[...], preferred_element_type=jnp.float32)
    hws_ref[...] = hw * dinv_ref[...]


def _layer2a_tc(h, stats, dinv, W2, g1, be1):
    return pl.pallas_call(
        _layer2a_body,
        grid=(_NBLK,),
        in_specs=[
            pl.BlockSpec((_BLK, HID), lambda i: (i, 0)),
            pl.BlockSpec((2, HID), lambda i: (0, 0)),
            pl.BlockSpec((_BLK, 1), lambda i: (i, 0)),
            pl.BlockSpec((HID, OUT), lambda i: (0, 0)),
            pl.BlockSpec((1, HID), lambda i: (0, 0)),
            pl.BlockSpec((1, HID), lambda i: (0, 0)),
        ],
        out_specs=pl.BlockSpec((_BLK, OUT), lambda i: (i, 0)),
        out_shape=jax.ShapeDtypeStruct((NPAD, OUT), jnp.float32),
    )(h, stats, dinv, W2, g1, be1)


def _final_body(p20_ref, p21_ref, hws_ref, dinv_ref, batch_ref, g2_ref, be2_ref,
                out_ref, ssum, ssq, pooled, cntf):
    i = pl.program_id(0)
    a = (p20_ref[...] + p21_ref[...] - hws_ref[...]) * dinv_ref[...]
    rows = i * _BLK + lax.broadcasted_iota(jnp.int32, (_BLK, 1), 0)
    mask = rows < N
    am = jnp.where(mask, a, 0.0)

    @pl.when(i == 0)
    def _():
        ssum[...] = jnp.zeros_like(ssum)
        ssq[...] = jnp.zeros_like(ssq)
        pooled[...] = jnp.zeros_like(pooled)
        cntf[...] = jnp.zeros_like(cntf)

    ssum[...] += jnp.sum(am, axis=0, keepdims=True)
    ssq[...] += jnp.sum(am * am, axis=0, keepdims=True)
    gids = lax.broadcasted_iota(jnp.int32, (_BLK, NG), 1)
    onehot = jnp.where((batch_ref[...] == gids) & mask, 1.0, 0.0)
    pooled[...] += lax.dot_general(onehot, am, (((0,), (0,)), ((), ())),
                                   preferred_element_type=jnp.float32)
    maskb = jnp.where(mask, 1.0, 0.0) * jnp.ones((_BLK, OUT), jnp.float32)
    cntf[...] += lax.dot_general(onehot, maskb, (((0,), (0,)), ((), ())),
                                 preferred_element_type=jnp.float32)

    @pl.when(i == _NBLK - 1)
    def _():
        mu = ssum[...] * (1.0 / N)
        var = ssq[...] * (1.0 / N) - mu * mu
        inv = lax.rsqrt(var + 1e-5)
        cnt = cntf[...]
        pm = pooled[...] / jnp.maximum(cnt, 1.0)
        res = (pm - mu) * inv * g2_ref[...] + be2_ref[...]
        out_ref[...] = jnp.where(cnt > 0.0, res, 0.0)


def _final_tc(p20, p21, hws, dinv, batchp, g2, be2):
    return pl.pallas_call(
        _final_body,
        grid=(_NBLK,),
        in_specs=[
            pl.BlockSpec((_BLK, OUT), lambda i: (i, 0)),
            pl.BlockSpec((_BLK, OUT), lambda i: (i, 0)),
            pl.BlockSpec((_BLK, OUT), lambda i: (i, 0)),
            pl.BlockSpec((_BLK, 1), lambda i: (i, 0)),
            pl.BlockSpec((_BLK, 1), lambda i: (i, 0)),
            pl.BlockSpec((1, OUT), lambda i: (0, 0)),
            pl.BlockSpec((1, OUT), lambda i: (0, 0)),
        ],
        out_specs=pl.BlockSpec((NG, OUT), lambda i: (0, 0)),
        out_shape=jax.ShapeDtypeStruct((NG, OUT), jnp.float32),
        scratch_shapes=[
            pltpu.VMEM((1, OUT), jnp.float32),
            pltpu.VMEM((1, OUT), jnp.float32),
            pltpu.VMEM((NG, OUT), jnp.float32),
            pltpu.VMEM((NG, OUT), jnp.float32),
        ],
    )(p20, p21, hws, dinv, batchp, g2, be2)


def kernel(x, edge_index, batch, W1, b1, gamma1, beta1, W2, b2, gamma2, beta2):
    del b2
    src = edge_index[0]
    dst = edge_index[1]
    pad_e = jnp.full((EPAD - E,), DUMP, jnp.int32)
    srcp = jnp.concatenate([src, pad_e]).reshape(ECH, CH)
    dstp = jnp.concatenate([dst, pad_e]).reshape(ECH, CH)
    x_pad = jnp.pad(x, ((0, NPAD - N), (0, 0)))
    batchp = jnp.pad(batch, (0, NPAD - N)).reshape(NPAD, 1)

    degp = _deg_sc(dstp)
    deg0 = degp[0].reshape(NPAD, 1)
    deg1 = degp[1].reshape(NPAD, 1)
    xsa, xsb, dinv = _prep_tc(deg0, deg1, x_pad)
    p1 = _agg1_sc(srcp, dstp, xsa, xsb)
    h, stats1 = _layer1_tc(p1[0], p1[1], dinv, W1, b1.reshape(1, HID))
    hws = _layer2a_tc(h, stats1, dinv, W2,
                      gamma1.reshape(1, HID), beta1.reshape(1, HID))
    p2 = _agg2_sc(srcp, dstp, hws)
    out = _final_tc(p2[0], p2[1], hws, dinv, batchp,
                    gamma2.reshape(1, OUT), beta2.reshape(1, OUT))
    return out


# speedup vs baseline: 7.3112x; 1.0323x over previous
import functools

import jax
import jax.numpy as jnp
from jax import lax
from jax.experimental import pallas as pl
from jax.experimental.pallas import tpu as pltpu
from jax.experimental.pallas import tpu_sc as plsc

N = 10000
NPAD = 10240
E = 160000
EPAD = 163840
IN_C = 256
HID = 1024
OUT = 128
NG = 16
DUMP = N

NC, NS = 2, 16
CH = 64
NBUF = 4
GRP = 40
ECH = EPAD // CH
SLAB = NPAD // NS

_SC_MESH = dict(mesh=plsc.VectorSubcoreMesh(core_axis_name="c", subcore_axis_name="s"))


@functools.partial(
    pl.kernel,
    out_type=jax.ShapeDtypeStruct((NC, NPAD), jnp.float32),
    scratch_types=[
        pltpu.VMEM_SHARED((NPAD,), jnp.float32),
        pltpu.VMEM((SLAB,), jnp.float32),
        pltpu.VMEM((CH,), jnp.float32),
        pltpu.VMEM((EPAD // (NC * NS) // CH, CH), jnp.int32),
    ]
    + [pltpu.SemaphoreType.DMA] * NBUF,
    **_SC_MESH,
)
def _deg_sc(dst_hbm, out_hbm, acc, zbuf, onesv, idxd, *ssems):
    c = lax.axis_index("c")
    s = lax.axis_index("s")
    per_tile = EPAD // (NC * NS)
    nch = per_tile // CH
    for k in range(SLAB // 16):
        zbuf[pl.ds(k * 16, 16)] = jnp.zeros((16,), jnp.float32)
    for k in range(CH // 16):
        onesv[pl.ds(k * 16, 16)] = jnp.ones((16,), jnp.float32)
    tile = c * NS + s
    pltpu.sync_copy(dst_hbm.at[pl.ds(tile * nch, nch)], idxd)
    pltpu.sync_copy(zbuf, acc.at[pl.ds(s * SLAB, SLAB)])
    plsc.subcore_barrier()

    def _wait_sc(b):
        pltpu.make_async_copy(onesv, acc.at[idxd.at[0]], ssems[b]).wait()

    for b in range(NBUF):
        pltpu.async_copy(onesv, acc.at[idxd.at[b]], ssems[b], add=True)

    @pl.loop(0, nch // NBUF - 1)
    def _(r):
        for b in range(NBUF):
            _wait_sc(b)
            pltpu.async_copy(onesv, acc.at[idxd.at[(r + 1) * NBUF + b]], ssems[b], add=True)

    for b in range(NBUF):
        _wait_sc(b)
    plsc.subcore_barrier()
    pltpu.sync_copy(acc.at[pl.ds(s * SLAB, SLAB)], out_hbm.at[c, pl.ds(s * SLAB, SLAB)])


def _agg_pipeline(table_hbm, src_hbm, dst_hbm, chunk0, nch,
                  idxs, idxd, rows, gsems, ssems, acc):
    def _wait_g(b):
        pltpu.make_async_copy(table_hbm.at[idxs.at[0]], rows[b], gsems[b]).wait()

    def _wait_s(b):
        pltpu.make_async_copy(rows[b], acc.at[idxd.at[0]], ssems[b]).wait()

    @pl.loop(0, nch // GRP)
    def _(g):
        pltpu.sync_copy(src_hbm.at[pl.ds(chunk0 + g * GRP, GRP)], idxs)
        pltpu.sync_copy(dst_hbm.at[pl.ds(chunk0 + g * GRP, GRP)], idxd)

        for b in range(NBUF):
            pltpu.async_copy(table_hbm.at[idxs.at[b]], rows[b], gsems[b])

        @pl.loop(0, GRP // NBUF)
        def _(r):
            for b in range(NBUF):
                _wait_g(b)
                pltpu.async_copy(rows[b], acc.at[idxd.at[r * NBUF + b]], ssems[b], add=True)
            for b in range(NBUF):
                nxt = (r + 1) * NBUF + b
                _wait_s(b)

                @pl.when(nxt < GRP)
                def _():
                    pltpu.async_copy(table_hbm.at[idxs.at[nxt]], rows[b], gsems[b])


_AGG_SCRATCH = [
    pltpu.VMEM_SHARED((NPAD, OUT), jnp.float32),
    pltpu.VMEM((GRP, CH), jnp.int32),
    pltpu.VMEM((GRP, CH), jnp.int32),
] + [pltpu.VMEM((CH, OUT), jnp.float32)] * NBUF + [pltpu.SemaphoreType.DMA] * (2 * NBUF)


_C_NCH = EPAD // NS // CH


@functools.partial(
    pl.kernel,
    out_type=jax.ShapeDtypeStruct((NC, NPAD, OUT), jnp.float32),
    scratch_types=_AGG_SCRATCH,
    **_SC_MESH,
)
def _agg1_sc(src_hbm, dst_hbm, xsa_hbm, xsb_hbm, out_hbm, acc, idxs, idxd, *bufs):
    c = lax.axis_index("c")
    s = lax.axis_index("s")
    rows = bufs[:NBUF]
    gsems = bufs[NBUF:2 * NBUF]
    ssems = bufs[2 * NBUF:]
    slab = pl.ds(s * SLAB, SLAB)

    @pl.when(c == 0)
    def _():
        pltpu.sync_copy(xsa_hbm.at[slab], acc.at[slab])

    @pl.when(c == 1)
    def _():
        pltpu.sync_copy(xsb_hbm.at[slab], acc.at[slab])

    plsc.subcore_barrier()

    @pl.when(c == 0)
    def _():
        _agg_pipeline(xsa_hbm, src_hbm, dst_hbm, s * _C_NCH, _C_NCH,
                      idxs, idxd, rows, gsems, ssems, acc)

    @pl.when(c == 1)
    def _():
        _agg_pipeline(xsb_hbm, src_hbm, dst_hbm, s * _C_NCH, _C_NCH,
                      idxs, idxd, rows, gsems, ssems, acc)

    plsc.subcore_barrier()
    pltpu.sync_copy(acc.at[slab], out_hbm.at[c, slab])


_F_NCH = EPAD // (NC * NS) // CH


@functools.partial(
    pl.kernel,
    out_type=jax.ShapeDtypeStruct((NC, NPAD, OUT), jnp.float32),
    scratch_types=_AGG_SCRATCH,
    **_SC_MESH,
)
def _agg2_sc(src_hbm, dst_hbm, hwsa_hbm, hwsb_hbm, out_hbm, acc, idxs, idxd, *bufs):
    c = lax.axis_index("c")
    s = lax.axis_index("s")
    rows = bufs[:NBUF]
    gsems = bufs[NBUF:2 * NBUF]
    ssems = bufs[2 * NBUF:]
    slab = pl.ds(s * SLAB, SLAB)
    tile = c * NS + s

    @pl.when(c == 0)
    def _():
        pltpu.sync_copy(hwsa_hbm.at[slab], acc.at[slab])

    @pl.when(c == 1)
    def _():
        pltpu.sync_copy(hwsb_hbm.at[slab], acc.at[slab])

    plsc.subcore_barrier()

    @pl.when(c == 0)
    def _():
        _agg_pipeline(hwsa_hbm, src_hbm, dst_hbm, tile * _F_NCH, _F_NCH,
                      idxs, idxd, rows, gsems, ssems, acc)

    @pl.when(c == 1)
    def _():
        _agg_pipeline(hwsb_hbm, src_hbm, dst_hbm, tile * _F_NCH, _F_NCH,
                      idxs, idxd, rows, gsems, ssems, acc)

    plsc.subcore_barrier()
    pltpu.sync_copy(acc.at[slab], out_hbm.at[c, slab])


_BLK = 512
_NBLK = NPAD // _BLK


def _prep_body(deg0_ref, deg1_ref, x_ref, xsa_ref, xsb_ref, dinv_ref):
    deg = deg0_ref[...] + deg1_ref[...] + 1.0
    dinv = lax.rsqrt(deg)
    dinv_ref[...] = dinv
    xs = x_ref[...] * dinv
    xsa_ref[...] = xs[:, :OUT]
    xsb_ref[...] = xs[:, OUT:]


def _prep_tc(deg0, deg1, x_pad):
    return pl.pallas_call(
        _prep_body,
        grid=(_NBLK,),
        in_specs=[
            pl.BlockSpec((_BLK, 1), lambda i: (i, 0)),
            pl.BlockSpec((_BLK, 1), lambda i: (i, 0)),
            pl.BlockSpec((_BLK, IN_C), lambda i: (i, 0)),
        ],
        out_specs=[
            pl.BlockSpec((_BLK, OUT), lambda i: (i, 0)),
            pl.BlockSpec((_BLK, OUT), lambda i: (i, 0)),
            pl.BlockSpec((_BLK, 1), lambda i: (i, 0)),
        ],
        out_shape=[
            jax.ShapeDtypeStruct((NPAD, OUT), jnp.float32),
            jax.ShapeDtypeStruct((NPAD, OUT), jnp.float32),
            jax.ShapeDtypeStruct((NPAD, 1), jnp.float32),
        ],
    )(deg0, deg1, x_pad)


def _layer1_body(pa_ref, pb_ref, dinv_ref, w1_ref, b1_ref, h_ref, stats_ref, ssum, ssq):
    i = pl.program_id(0)
    dv = dinv_ref[...]
    pa = pa_ref[...] * dv
    pb = pb_ref[...] * dv
    w1 = w1_ref[...]
    h = (
        jnp.dot(pa, w1[:OUT, :], preferred_element_type=jnp.float32)
        + jnp.dot(pb, w1[OUT:, :], preferred_element_type=jnp.float32)
        + b1_ref[...]
    )
    h_ref[...] = h
    rows = i * _BLK + lax.broadcasted_iota(jnp.int32, (_BLK, 1), 0)
    hm = jnp.where(rows < N, h, 0.0)

    @pl.when(i == 0)
    def _():
        ssum[...] = jnp.zeros_like(ssum)
        ssq[...] = jnp.zeros_like(ssq)

    ssum[...] += jnp.sum(hm, axis=0, keepdims=True)
    ssq[...] += jnp.sum(hm * hm, axis=0, keepdims=True)

    @pl.when(i == _NBLK - 1)
    def _():
        stats_ref[...] = jnp.concatenate([ssum[...], ssq[...]], axis=0)


def _layer1_tc(pa, pb, dinv, W1, b1):
    return pl.pallas_call(
        _layer1_body,
        grid=(_NBLK,),
        in_specs=[
            pl.BlockSpec((_BLK, OUT), lambda i: (i, 0)),
            pl.BlockSpec((_BLK, OUT), lambda i: (i, 0)),
            pl.BlockSpec((_BLK, 1), lambda i: (i, 0)),
            pl.BlockSpec((IN_C, HID), lambda i: (0, 0)),
            pl.BlockSpec((1, HID), lambda i: (0, 0)),
        ],
        out_specs=[
            pl.BlockSpec((_BLK, HID), lambda i: (i, 0)),
            pl.BlockSpec((2, HID), lambda i: (0, 0)),
        ],
        out_shape=[
            jax.ShapeDtypeStruct((NPAD, HID), jnp.float32),
            jax.ShapeDtypeStruct((2, HID), jnp.float32),
        ],
        scratch_shapes=[
            pltpu.VMEM((1, HID), jnp.float32),
            pltpu.VMEM((1, HID), jnp.float32),
        ],
    )(pa, pb, dinv, W1, b1)


def _layer2a_body(h_ref, stats_ref, dinv_ref, w2_ref, g1_ref, be1_ref,
                  hwsa_ref, hwsb_ref):
    stats = stats_ref[...]
    mu = stats[0:1, :] * (1.0 / N)
    var = stats[1:2, :] * (1.0 / N) - mu * mu
    alpha = g1_ref[...] * lax.rsqrt(var + 1e-5)
    c0 = be1_ref[...] - mu * alpha
    hn = jnp.maximum(h_ref[...] * alpha + c0, 0.0)
    hw = jnp.dot(hn, w2_ref[...], preferred_element_type=jnp.float32)
    hws = hw * dinv_ref[...]
    hwsa_ref[...] = hws
    hwsb_ref[...] = hws


def _layer2a_tc(h, stats, dinv, W2, g1, be1):
    return pl.pallas_call(
        _layer2a_body,
        grid=(_NBLK,),
        in_specs=[
            pl.BlockSpec((_BLK, HID), lambda i: (i, 0)),
            pl.BlockSpec((2, HID), lambda i: (0, 0)),
            pl.BlockSpec((_BLK, 1), lambda i: (i, 0)),
            pl.BlockSpec((HID, OUT), lambda i: (0, 0)),
            pl.BlockSpec((1, HID), lambda i: (0, 0)),
            pl.BlockSpec((1, HID), lambda i: (0, 0)),
        ],
        out_specs=[
            pl.BlockSpec((_BLK, OUT), lambda i: (i, 0)),
            pl.BlockSpec((_BLK, OUT), lambda i: (i, 0)),
        ],
        out_shape=[
            jax.ShapeDtypeStruct((NPAD, OUT), jnp.float32),
            jax.ShapeDtypeStruct((NPAD, OUT), jnp.float32),
        ],
    )(h, stats, dinv, W2, g1, be1)


def _final_body(p20_ref, p21_ref, hws_ref, dinv_ref, batch_ref, g2_ref, be2_ref,
                out_ref, ssum, ssq, pooled, cntf):
    i = pl.program_id(0)
    a = (p20_ref[...] + p21_ref[...] - hws_ref[...]) * dinv_ref[...]
    rows = i * _BLK + lax.broadcasted_iota(jnp.int32, (_BLK, 1), 0)
    mask = rows < N
    am = jnp.where(mask, a, 0.0)

    @pl.when(i == 0)
    def _():
        ssum[...] = jnp.zeros_like(ssum)
        ssq[...] = jnp.zeros_like(ssq)
        pooled[...] = jnp.zeros_like(pooled)
        cntf[...] = jnp.zeros_like(cntf)

    ssum[...] += jnp.sum(am, axis=0, keepdims=True)
    ssq[...] += jnp.sum(am * am, axis=0, keepdims=True)
    gids = lax.broadcasted_iota(jnp.int32, (_BLK, NG), 1)
    onehot = jnp.where((batch_ref[...] == gids) & mask, 1.0, 0.0)
    pooled[...] += lax.dot_general(onehot, am, (((0,), (0,)), ((), ())),
                                   preferred_element_type=jnp.float32)
    maskb = jnp.where(mask, 1.0, 0.0) * jnp.ones((_BLK, OUT), jnp.float32)
    cntf[...] += lax.dot_general(onehot, maskb, (((0,), (0,)), ((), ())),
                                 preferred_element_type=jnp.float32)

    @pl.when(i == _NBLK - 1)
    def _():
        mu = ssum[...] * (1.0 / N)
        var = ssq[...] * (1.0 / N) - mu * mu
        inv = lax.rsqrt(var + 1e-5)
        cnt = cntf[...]
        pm = pooled[...] / jnp.maximum(cnt, 1.0)
        res = (pm - mu) * inv * g2_ref[...] + be2_ref[...]
        out_ref[...] = jnp.where(cnt > 0.0, res, 0.0)


def _final_tc(p20, p21, hws, dinv, batchp, g2, be2):
    return pl.pallas_call(
        _final_body,
        grid=(_NBLK,),
        in_specs=[
            pl.BlockSpec((_BLK, OUT), lambda i: (i, 0)),
            pl.BlockSpec((_BLK, OUT), lambda i: (i, 0)),
            pl.BlockSpec((_BLK, OUT), lambda i: (i, 0)),
            pl.BlockSpec((_BLK, 1), lambda i: (i, 0)),
            pl.BlockSpec((_BLK, 1), lambda i: (i, 0)),
            pl.BlockSpec((1, OUT), lambda i: (0, 0)),
            pl.BlockSpec((1, OUT), lambda i: (0, 0)),
        ],
        out_specs=pl.BlockSpec((NG, OUT), lambda i: (0, 0)),
        out_shape=jax.ShapeDtypeStruct((NG, OUT), jnp.float32),
        scratch_shapes=[
            pltpu.VMEM((1, OUT), jnp.float32),
            pltpu.VMEM((1, OUT), jnp.float32),
            pltpu.VMEM((NG, OUT), jnp.float32),
            pltpu.VMEM((NG, OUT), jnp.float32),
        ],
    )(p20, p21, hws, dinv, batchp, g2, be2)


def kernel(x, edge_index, batch, W1, b1, gamma1, beta1, W2, b2, gamma2, beta2):
    del b2
    src = edge_index[0]
    dst = edge_index[1]
    pad_e = jnp.full((EPAD - E,), DUMP, jnp.int32)
    srcp = jnp.concatenate([src, pad_e]).reshape(ECH, CH)
    dstp = jnp.concatenate([dst, pad_e]).reshape(ECH, CH)
    x_pad = jnp.pad(x, ((0, NPAD - N), (0, 0)))
    batchp = jnp.pad(batch, (0, NPAD - N)).reshape(NPAD, 1)

    degp = _deg_sc(dstp)
    deg0 = degp[0].reshape(NPAD, 1)
    deg1 = degp[1].reshape(NPAD, 1)
    xsa, xsb, dinv = _prep_tc(deg0, deg1, x_pad)
    p1 = _agg1_sc(srcp, dstp, xsa, xsb)
    h, stats1 = _layer1_tc(p1[0], p1[1], dinv, W1, b1.reshape(1, HID))
    hwsa, hwsb = _layer2a_tc(h, stats1, dinv, W2,
                             gamma1.reshape(1, HID), beta1.reshape(1, HID))
    p2 = _agg2_sc(srcp, dstp, hwsa, hwsb)
    out = _final_tc(p2[0], p2[1], hwsa, dinv, batchp,
                    gamma2.reshape(1, OUT), beta2.reshape(1, OUT))
    return out


# speedup vs baseline: 15.8119x; 2.1627x over previous
import functools

import jax
import jax.numpy as jnp
from jax import lax
from jax.experimental import pallas as pl
from jax.experimental.pallas import tpu as pltpu
from jax.experimental.pallas import tpu_sc as plsc

N = 10000
NPAD = 10240
E = 160000
EPAD = 163840
IN_C = 256
HID = 1024
OUT = 128
NG = 16
DUMP = N

NC, NS = 2, 16
CH = 64
NBUF = 4
GRP = 40
ECH = EPAD // CH
SLAB = NPAD // NS

_SC_MESH = dict(mesh=plsc.VectorSubcoreMesh(core_axis_name="c", subcore_axis_name="s"))


@functools.partial(
    pl.kernel,
    out_type=jax.ShapeDtypeStruct((NC, NPAD), jnp.float32),
    scratch_types=[
        pltpu.VMEM_SHARED((NPAD,), jnp.float32),
        pltpu.VMEM((SLAB,), jnp.float32),
        pltpu.VMEM((CH,), jnp.float32),
        pltpu.VMEM((EPAD // (NC * NS) // CH, CH), jnp.int32),
    ]
    + [pltpu.SemaphoreType.DMA] * NBUF,
    **_SC_MESH,
)
def _deg_sc(dst_hbm, out_hbm, acc, zbuf, onesv, idxd, *ssems):
    c = lax.axis_index("c")
    s = lax.axis_index("s")
    per_tile = EPAD // (NC * NS)
    nch = per_tile // CH
    for k in range(SLAB // 16):
        zbuf[pl.ds(k * 16, 16)] = jnp.zeros((16,), jnp.float32)
    for k in range(CH // 16):
        onesv[pl.ds(k * 16, 16)] = jnp.ones((16,), jnp.float32)
    tile = c * NS + s
    pltpu.sync_copy(dst_hbm.at[pl.ds(tile * nch, nch)], idxd)
    pltpu.sync_copy(zbuf, acc.at[pl.ds(s * SLAB, SLAB)])
    plsc.subcore_barrier()

    def _wait_sc(b):
        pltpu.make_async_copy(onesv, acc.at[idxd.at[0]], ssems[b]).wait()

    for b in range(NBUF):
        pltpu.async_copy(onesv, acc.at[idxd.at[b]], ssems[b], add=True)

    @pl.loop(0, nch // NBUF - 1)
    def _(r):
        for b in range(NBUF):
            _wait_sc(b)
            pltpu.async_copy(onesv, acc.at[idxd.at[(r + 1) * NBUF + b]], ssems[b], add=True)

    for b in range(NBUF):
        _wait_sc(b)
    plsc.subcore_barrier()
    pltpu.sync_copy(acc.at[pl.ds(s * SLAB, SLAB)], out_hbm.at[c, pl.ds(s * SLAB, SLAB)])


def _agg_pipeline(table_hbm, src_hbm, dst_hbm, chunk0, nch,
                  idxs, idxd, rows, gsems, ssems, acc):
    def _wait_g(b):
        pltpu.make_async_copy(table_hbm.at[idxs.at[0]], rows[b], gsems[b]).wait()

    def _wait_s(b):
        pltpu.make_async_copy(rows[b], acc.at[idxd.at[0]], ssems[b]).wait()

    @pl.loop(0, nch // GRP)
    def _(g):
        pltpu.sync_copy(src_hbm.at[pl.ds(chunk0 + g * GRP, GRP)], idxs)
        pltpu.sync_copy(dst_hbm.at[pl.ds(chunk0 + g * GRP, GRP)], idxd)

        for b in range(NBUF):
            pltpu.async_copy(table_hbm.at[idxs.at[b]], rows[b], gsems[b])

        @pl.loop(0, GRP // NBUF)
        def _(r):
            for b in range(NBUF):
                _wait_g(b)
                pltpu.async_copy(rows[b], acc.at[idxd.at[r * NBUF + b]], ssems[b], add=True)
            for b in range(NBUF):
                nxt = (r + 1) * NBUF + b
                _wait_s(b)

                @pl.when(nxt < GRP)
                def _():
                    pltpu.async_copy(table_hbm.at[idxs.at[nxt]], rows[b], gsems[b])


_AGG_SCRATCH = [
    pltpu.VMEM_SHARED((NPAD, OUT), jnp.float32),
    pltpu.VMEM((GRP, CH), jnp.int32),
    pltpu.VMEM((GRP, CH), jnp.int32),
] + [pltpu.VMEM((CH, OUT), jnp.float32)] * NBUF + [pltpu.SemaphoreType.DMA] * (2 * NBUF)


_C_NCH = EPAD // NS // CH


@functools.partial(
    pl.kernel,
    out_type=jax.ShapeDtypeStruct((NC, NPAD, OUT), jnp.float32),
    scratch_types=_AGG_SCRATCH,
    **_SC_MESH,
)
def _agg1_sc(src_hbm, dst_hbm, xsa_hbm, xsb_hbm, out_hbm, acc, idxs, idxd, *bufs):
    c = lax.axis_index("c")
    s = lax.axis_index("s")
    rows = bufs[:NBUF]
    gsems = bufs[NBUF:2 * NBUF]
    ssems = bufs[2 * NBUF:]
    slab = pl.ds(s * SLAB, SLAB)

    @pl.when(c == 0)
    def _():
        pltpu.sync_copy(xsa_hbm.at[slab], acc.at[slab])

    @pl.when(c == 1)
    def _():
        pltpu.sync_copy(xsb_hbm.at[slab], acc.at[slab])

    plsc.subcore_barrier()

    @pl.when(c == 0)
    def _():
        _agg_pipeline(xsa_hbm, src_hbm, dst_hbm, s * _C_NCH, _C_NCH,
                      idxs, idxd, rows, gsems, ssems, acc)

    @pl.when(c == 1)
    def _():
        _agg_pipeline(xsb_hbm, src_hbm, dst_hbm, s * _C_NCH, _C_NCH,
                      idxs, idxd, rows, gsems, ssems, acc)

    plsc.subcore_barrier()
    pltpu.sync_copy(acc.at[slab], out_hbm.at[c, slab])


_F_NCH = EPAD // (NC * NS) // CH


@functools.partial(
    pl.kernel,
    out_type=jax.ShapeDtypeStruct((NC, NPAD, OUT), jnp.float32),
    scratch_types=_AGG_SCRATCH,
    **_SC_MESH,
)
def _agg2_sc(src_hbm, dst_hbm, hwsa_hbm, hwsb_hbm, out_hbm, acc, idxs, idxd, *bufs):
    c = lax.axis_index("c")
    s = lax.axis_index("s")
    rows = bufs[:NBUF]
    gsems = bufs[NBUF:2 * NBUF]
    ssems = bufs[2 * NBUF:]
    slab = pl.ds(s * SLAB, SLAB)
    tile = c * NS + s

    @pl.when(c == 0)
    def _():
        pltpu.sync_copy(hwsa_hbm.at[slab], acc.at[slab])

    @pl.when(c == 1)
    def _():
        pltpu.sync_copy(hwsb_hbm.at[slab], acc.at[slab])

    plsc.subcore_barrier()

    @pl.when(c == 0)
    def _():
        _agg_pipeline(hwsa_hbm, src_hbm, dst_hbm, tile * _F_NCH, _F_NCH,
                      idxs, idxd, rows, gsems, ssems, acc)

    @pl.when(c == 1)
    def _():
        _agg_pipeline(hwsb_hbm, src_hbm, dst_hbm, tile * _F_NCH, _F_NCH,
                      idxs, idxd, rows, gsems, ssems, acc)

    plsc.subcore_barrier()
    pltpu.sync_copy(acc.at[slab], out_hbm.at[c, slab])


_BLK = 512
_NBLK = NPAD // _BLK


def _prep_body(deg0_ref, deg1_ref, x_ref, xsa_ref, xsb_ref, dinv_ref):
    deg = deg0_ref[...] + deg1_ref[...] + 1.0
    dinv = lax.rsqrt(deg)
    dinv_ref[...] = dinv
    xs = x_ref[...] * dinv
    xsa_ref[...] = xs[:, :OUT]
    xsb_ref[...] = xs[:, OUT:]


def _prep_tc(deg0, deg1, x_pad):
    return pl.pallas_call(
        _prep_body,
        grid=(_NBLK,),
        in_specs=[
            pl.BlockSpec((_BLK, 1), lambda i: (i, 0)),
            pl.BlockSpec((_BLK, 1), lambda i: (i, 0)),
            pl.BlockSpec((_BLK, IN_C), lambda i: (i, 0)),
        ],
        out_specs=[
            pl.BlockSpec((_BLK, OUT), lambda i: (i, 0)),
            pl.BlockSpec((_BLK, OUT), lambda i: (i, 0)),
            pl.BlockSpec((_BLK, 1), lambda i: (i, 0)),
        ],
        out_shape=[
            jax.ShapeDtypeStruct((NPAD, OUT), jnp.float32),
            jax.ShapeDtypeStruct((NPAD, OUT), jnp.float32),
            jax.ShapeDtypeStruct((NPAD, 1), jnp.float32),
        ],
    )(deg0, deg1, x_pad)


def _layer1_body(pa_ref, pb_ref, dinv_ref, w1_ref, b1_ref, h_ref, stats_ref, ssum, ssq):
    i = pl.program_id(0)
    dv = dinv_ref[...]
    pa = pa_ref[...] * dv
    pb = pb_ref[...] * dv
    w1 = w1_ref[...]
    h = (
        jnp.dot(pa, w1[:OUT, :], preferred_element_type=jnp.float32)
        + jnp.dot(pb, w1[OUT:, :], preferred_element_type=jnp.float32)
        + b1_ref[...]
    )
    h_ref[...] = h
    rows = i * _BLK + lax.broadcasted_iota(jnp.int32, (_BLK, 1), 0)
    hm = jnp.where(rows < N, h, 0.0)

    @pl.when(i == 0)
    def _():
        ssum[...] = jnp.zeros_like(ssum)
        ssq[...] = jnp.zeros_like(ssq)

    ssum[...] += jnp.sum(hm, axis=0, keepdims=True)
    ssq[...] += jnp.sum(hm * hm, axis=0, keepdims=True)

    @pl.when(i == _NBLK - 1)
    def _():
        stats_ref[...] = jnp.concatenate([ssum[...], ssq[...]], axis=0)


def _layer1_tc(pa, pb, dinv, W1, b1):
    return pl.pallas_call(
        _layer1_body,
        grid=(_NBLK,),
        in_specs=[
            pl.BlockSpec((_BLK, OUT), lambda i: (i, 0)),
            pl.BlockSpec((_BLK, OUT), lambda i: (i, 0)),
            pl.BlockSpec((_BLK, 1), lambda i: (i, 0)),
            pl.BlockSpec((IN_C, HID), lambda i: (0, 0)),
            pl.BlockSpec((1, HID), lambda i: (0, 0)),
        ],
        out_specs=[
            pl.BlockSpec((_BLK, HID), lambda i: (i, 0)),
            pl.BlockSpec((2, HID), lambda i: (0, 0)),
        ],
        out_shape=[
            jax.ShapeDtypeStruct((NPAD, HID), jnp.float32),
            jax.ShapeDtypeStruct((2, HID), jnp.float32),
        ],
        scratch_shapes=[
            pltpu.VMEM((1, HID), jnp.float32),
            pltpu.VMEM((1, HID), jnp.float32),
        ],
    )(pa, pb, dinv, W1, b1)


def _layer2a_body(h_ref, stats_ref, dinv_ref, w2_ref, g1_ref, be1_ref,
                  hwsa_ref, hwsb_ref):
    stats = stats_ref[...]
    mu = stats[0:1, :] * (1.0 / N)
    var = stats[1:2, :] * (1.0 / N) - mu * mu
    alpha = g1_ref[...] * lax.rsqrt(var + 1e-5)
    c0 = be1_ref[...] - mu * alpha
    hn = jnp.maximum(h_ref[...] * alpha + c0, 0.0)
    hw = jnp.dot(hn, w2_ref[...], preferred_element_type=jnp.float32)
    hws = hw * dinv_ref[...]
    hwsa_ref[...] = hws
    hwsb_ref[...] = hws


def _layer2a_tc(h, stats, dinv, W2, g1, be1):
    return pl.pallas_call(
        _layer2a_body,
        grid=(_NBLK,),
        in_specs=[
            pl.BlockSpec((_BLK, HID), lambda i: (i, 0)),
            pl.BlockSpec((2, HID), lambda i: (0, 0)),
            pl.BlockSpec((_BLK, 1), lambda i: (i, 0)),
            pl.BlockSpec((HID, OUT), lambda i: (0, 0)),
            pl.BlockSpec((1, HID), lambda i: (0, 0)),
            pl.BlockSpec((1, HID), lambda i: (0, 0)),
        ],
        out_specs=[
            pl.BlockSpec((_BLK, OUT), lambda i: (i, 0)),
            pl.BlockSpec((_BLK, OUT), lambda i: (i, 0)),
        ],
        out_shape=[
            jax.ShapeDtypeStruct((NPAD, OUT), jnp.float32),
            jax.ShapeDtypeStruct((NPAD, OUT), jnp.float32),
        ],
    )(h, stats, dinv, W2, g1, be1)


def _final_body(p20_ref, p21_ref, hws_ref, dinv_ref, batch_ref, g2_ref, be2_ref,
                out_ref, ssum, ssq, pooled, cntf):
    i = pl.program_id(0)
    a = (p20_ref[...] + p21_ref[...] - hws_ref[...]) * dinv_ref[...]
    rows = i * _BLK + lax.broadcasted_iota(jnp.int32, (_BLK, 1), 0)
    mask = rows < N
    am = jnp.where(mask, a, 0.0)

    @pl.when(i == 0)
    def _():
        ssum[...] = jnp.zeros_like(ssum)
        ssq[...] = jnp.zeros_like(ssq)
        pooled[...] = jnp.zeros_like(pooled)
        cntf[...] = jnp.zeros_like(cntf)

    ssum[...] += jnp.sum(am, axis=0, keepdims=True)
    ssq[...] += jnp.sum(am * am, axis=0, keepdims=True)
    gids = lax.broadcasted_iota(jnp.int32, (_BLK, NG), 1)
    onehot = jnp.where((batch_ref[...] == gids) & mask, 1.0, 0.0)
    pooled[...] += lax.dot_general(onehot, am, (((0,), (0,)), ((), ())),
                                   preferred_element_type=jnp.float32)
    maskb = jnp.where(mask, 1.0, 0.0) * jnp.ones((_BLK, OUT), jnp.float32)
    cntf[...] += lax.dot_general(onehot, maskb, (((0,), (0,)), ((), ())),
                                 preferred_element_type=jnp.float32)

    @pl.when(i == _NBLK - 1)
    def _():
        mu = ssum[...] * (1.0 / N)
        var = ssq[...] * (1.0 / N) - mu * mu
        inv = lax.rsqrt(var + 1e-5)
        cnt = cntf[...]
        pm = pooled[...] / jnp.maximum(cnt, 1.0)
        res = (pm - mu) * inv * g2_ref[...] + be2_ref[...]
        out_ref[...] = jnp.where(cnt > 0.0, res, 0.0)


def _final_tc(p20, p21, hws, dinv, batchp, g2, be2):
    return pl.pallas_call(
        _final_body,
        grid=(_NBLK,),
        in_specs=[
            pl.BlockSpec((_BLK, OUT), lambda i: (i, 0)),
            pl.BlockSpec((_BLK, OUT), lambda i: (i, 0)),
            pl.BlockSpec((_BLK, OUT), lambda i: (i, 0)),
            pl.BlockSpec((_BLK, 1), lambda i: (i, 0)),
            pl.BlockSpec((_BLK, 1), lambda i: (i, 0)),
            pl.BlockSpec((1, OUT), lambda i: (0, 0)),
            pl.BlockSpec((1, OUT), lambda i: (0, 0)),
        ],
        out_specs=pl.BlockSpec((NG, OUT), lambda i: (0, 0)),
        out_shape=jax.ShapeDtypeStruct((NG, OUT), jnp.float32),
        scratch_shapes=[
            pltpu.VMEM((1, OUT), jnp.float32),
            pltpu.VMEM((1, OUT), jnp.float32),
            pltpu.VMEM((NG, OUT), jnp.float32),
            pltpu.VMEM((NG, OUT), jnp.float32),
        ],
    )(p20, p21, hws, dinv, batchp, g2, be2)


def kernel(x, edge_index, batch, W1, b1, gamma1, beta1, W2, b2, gamma2, beta2):
    del b2
    src = edge_index[0]
    dst = edge_index[1]
    pad_e = DUMP + jnp.arange(EPAD - E, dtype=jnp.int32) % (NPAD - N)
    srcp = jnp.concatenate([src, pad_e]).reshape(ECH, CH)
    dstp = jnp.concatenate([dst, pad_e]).reshape(ECH, CH)
    x_pad = jnp.pad(x, ((0, NPAD - N), (0, 0)))
    batchp = jnp.pad(batch, (0, NPAD - N)).reshape(NPAD, 1)

    degp = _deg_sc(dstp)
    deg0 = degp[0].reshape(NPAD, 1)
    deg1 = degp[1].reshape(NPAD, 1)
    xsa, xsb, dinv = _prep_tc(deg0, deg1, x_pad)
    p1 = _agg1_sc(srcp, dstp, xsa, xsb)
    h, stats1 = _layer1_tc(p1[0], p1[1], dinv, W1, b1.reshape(1, HID))
    hwsa, hwsb = _layer2a_tc(h, stats1, dinv, W2,
                             gamma1.reshape(1, HID), beta1.reshape(1, HID))
    p2 = _agg2_sc(srcp, dstp, hwsa, hwsb)
    out = _final_tc(p2[0], p2[1], hwsa, dinv, batchp,
                    gamma2.reshape(1, OUT), beta2.reshape(1, OUT))
    return out


# speedup vs baseline: 16.6039x; 1.0501x over previous
import functools

import jax
import jax.numpy as jnp
from jax import lax
from jax.experimental import pallas as pl
from jax.experimental.pallas import tpu as pltpu
from jax.experimental.pallas import tpu_sc as plsc

N = 10000
NPAD = 10240
E = 160000
EPAD = 163840
IN_C = 256
HID = 1024
OUT = 128
NG = 16
DUMP = N

NC, NS = 2, 16
CH = 64
NBUF = 4
GRP = 40
ECH = EPAD // CH
SLAB = NPAD // NS

_SC_MESH = dict(mesh=plsc.VectorSubcoreMesh(core_axis_name="c", subcore_axis_name="s"))


@functools.partial(
    pl.kernel,
    out_type=jax.ShapeDtypeStruct((NC, NPAD), jnp.float32),
    scratch_types=[
        pltpu.VMEM_SHARED((NPAD,), jnp.float32),
        pltpu.VMEM((SLAB,), jnp.float32),
        pltpu.VMEM((CH,), jnp.float32),
        pltpu.VMEM((EPAD // (NC * NS) // CH, CH), jnp.int32),
    ]
    + [pltpu.SemaphoreType.DMA] * NBUF,
    **_SC_MESH,
)
def _deg_sc(dst_hbm, out_hbm, acc, zbuf, onesv, idxd, *ssems):
    c = lax.axis_index("c")
    s = lax.axis_index("s")
    per_tile = EPAD // (NC * NS)
    nch = per_tile // CH
    for k in range(SLAB // 16):
        zbuf[pl.ds(k * 16, 16)] = jnp.zeros((16,), jnp.float32)
    for k in range(CH // 16):
        onesv[pl.ds(k * 16, 16)] = jnp.ones((16,), jnp.float32)
    tile = c * NS + s
    pltpu.sync_copy(dst_hbm.at[pl.ds(tile * nch, nch)], idxd)
    pltpu.sync_copy(zbuf, acc.at[pl.ds(s * SLAB, SLAB)])
    plsc.subcore_barrier()

    def _wait_sc(b):
        pltpu.make_async_copy(onesv, acc.at[idxd.at[0]], ssems[b]).wait()

    for b in range(NBUF):
        pltpu.async_copy(onesv, acc.at[idxd.at[b]], ssems[b], add=True)

    @pl.loop(0, nch // NBUF - 1)
    def _(r):
        for b in range(NBUF):
            _wait_sc(b)
            pltpu.async_copy(onesv, acc.at[idxd.at[(r + 1) * NBUF + b]], ssems[b], add=True)

    for b in range(NBUF):
        _wait_sc(b)
    plsc.subcore_barrier()
    pltpu.sync_copy(acc.at[pl.ds(s * SLAB, SLAB)], out_hbm.at[c, pl.ds(s * SLAB, SLAB)])


def _agg_pipeline(table_hbm, src_hbm, dst_hbm, chunk0, nch,
                  idxs, idxd, rows, gsems, ssems, acc):
    def _wait_g(b):
        pltpu.make_async_copy(table_hbm.at[idxs.at[0]], rows[b], gsems[b]).wait()

    def _wait_s(b):
        pltpu.make_async_copy(rows[b], acc.at[idxd.at[0]], ssems[b]).wait()

    @pl.loop(0, nch // GRP)
    def _(g):
        pltpu.sync_copy(src_hbm.at[pl.ds(chunk0 + g * GRP, GRP)], idxs)
        pltpu.sync_copy(dst_hbm.at[pl.ds(chunk0 + g * GRP, GRP)], idxd)

        for b in range(NBUF):
            pltpu.async_copy(table_hbm.at[idxs.at[b]], rows[b], gsems[b])

        @pl.loop(0, GRP // NBUF)
        def _(r):
            for b in range(NBUF):
                _wait_g(b)
                pltpu.async_copy(rows[b], acc.at[idxd.at[r * NBUF + b]], ssems[b], add=True)
            for b in range(NBUF):
                nxt = (r + 1) * NBUF + b
                _wait_s(b)

                @pl.when(nxt < GRP)
                def _():
                    pltpu.async_copy(table_hbm.at[idxs.at[nxt]], rows[b], gsems[b])


_AGG_SCRATCH = [
    pltpu.VMEM_SHARED((NPAD, OUT), jnp.float32),
    pltpu.VMEM((GRP, CH), jnp.int32),
    pltpu.VMEM((GRP, CH), jnp.int32),
] + [pltpu.VMEM((CH, OUT), jnp.float32)] * NBUF + [pltpu.SemaphoreType.DMA] * (2 * NBUF)


_C_NCH = EPAD // NS // CH


@functools.partial(
    pl.kernel,
    out_type=jax.ShapeDtypeStruct((NC, NPAD, OUT), jnp.float32),
    scratch_types=_AGG_SCRATCH,
    **_SC_MESH,
)
def _agg1_sc(src_hbm, dst_hbm, xsa_hbm, xsb_hbm, out_hbm, acc, idxs, idxd, *bufs):
    c = lax.axis_index("c")
    s = lax.axis_index("s")
    rows = bufs[:NBUF]
    gsems = bufs[NBUF:2 * NBUF]
    ssems = bufs[2 * NBUF:]
    slab = pl.ds(s * SLAB, SLAB)

    @pl.when(c == 0)
    def _():
        pltpu.sync_copy(xsa_hbm.at[slab], acc.at[slab])

    @pl.when(c == 1)
    def _():
        pltpu.sync_copy(xsb_hbm.at[slab], acc.at[slab])

    plsc.subcore_barrier()

    @pl.when(c == 0)
    def _():
        _agg_pipeline(xsa_hbm, src_hbm, dst_hbm, s * _C_NCH, _C_NCH,
                      idxs, idxd, rows, gsems, ssems, acc)

    @pl.when(c == 1)
    def _():
        _agg_pipeline(xsb_hbm, src_hbm, dst_hbm, s * _C_NCH, _C_NCH,
                      idxs, idxd, rows, gsems, ssems, acc)

    plsc.subcore_barrier()
    pltpu.sync_copy(acc.at[slab], out_hbm.at[c, slab])


_F_NCH = EPAD // (NC * NS) // CH


@functools.partial(
    pl.kernel,
    out_type=jax.ShapeDtypeStruct((NC, NPAD, OUT), jnp.float32),
    scratch_types=_AGG_SCRATCH,
    **_SC_MESH,
)
def _agg2_sc(src_hbm, dst_hbm, hwsa_hbm, hwsb_hbm, out_hbm, acc, idxs, idxd, *bufs):
    c = lax.axis_index("c")
    s = lax.axis_index("s")
    rows = bufs[:NBUF]
    gsems = bufs[NBUF:2 * NBUF]
    ssems = bufs[2 * NBUF:]
    slab = pl.ds(s * SLAB, SLAB)
    tile = c * NS + s

    @pl.when(c == 0)
    def _():
        pltpu.sync_copy(hwsa_hbm.at[slab], acc.at[slab])

    @pl.when(c == 1)
    def _():
        pltpu.sync_copy(hwsb_hbm.at[slab], acc.at[slab])

    plsc.subcore_barrier()

    @pl.when(c == 0)
    def _():
        _agg_pipeline(hwsa_hbm, src_hbm, dst_hbm, tile * _F_NCH, _F_NCH,
                      idxs, idxd, rows, gsems, ssems, acc)

    @pl.when(c == 1)
    def _():
        _agg_pipeline(hwsb_hbm, src_hbm, dst_hbm, tile * _F_NCH, _F_NCH,
                      idxs, idxd, rows, gsems, ssems, acc)

    plsc.subcore_barrier()
    pltpu.sync_copy(acc.at[slab], out_hbm.at[c, slab])


_BLK = 512
_NBLK = NPAD // _BLK


def _prep_body(deg0_ref, deg1_ref, x_ref, xsa_ref, xsb_ref, dinv_ref):
    deg = deg0_ref[...] + deg1_ref[...] + 1.0
    dinv = lax.rsqrt(deg)
    dinv_ref[...] = dinv
    xs = x_ref[...] * dinv
    xsa_ref[...] = xs[:, :OUT]
    xsb_ref[...] = xs[:, OUT:]


def _prep_tc(deg0, deg1, x_pad):
    return pl.pallas_call(
        _prep_body,
        grid=(_NBLK,),
        in_specs=[
            pl.BlockSpec((_BLK, 1), lambda i: (i, 0)),
            pl.BlockSpec((_BLK, 1), lambda i: (i, 0)),
            pl.BlockSpec((_BLK, IN_C), lambda i: (i, 0)),
        ],
        out_specs=[
            pl.BlockSpec((_BLK, OUT), lambda i: (i, 0)),
            pl.BlockSpec((_BLK, OUT), lambda i: (i, 0)),
            pl.BlockSpec((_BLK, 1), lambda i: (i, 0)),
        ],
        out_shape=[
            jax.ShapeDtypeStruct((NPAD, OUT), jnp.float32),
            jax.ShapeDtypeStruct((NPAD, OUT), jnp.float32),
            jax.ShapeDtypeStruct((NPAD, 1), jnp.float32),
        ],
    )(deg0, deg1, x_pad)


def _layers_body(pa_ref, pb_ref, dinv_ref, w1_ref, b1_ref, w2_ref, g1_ref,
                 be1_ref, hwsa_ref, hwsb_ref, h_scr, ssum, ssq):
    i = pl.program_id(0)

    @pl.when(i < _NBLK)
    def _():
        dv = dinv_ref[...]
        pa = pa_ref[...] * dv
        pb = pb_ref[...] * dv
        w1 = w1_ref[...]
        h = (
            jnp.dot(pa, w1[:OUT, :], preferred_element_type=jnp.float32)
            + jnp.dot(pb, w1[OUT:, :], preferred_element_type=jnp.float32)
            + b1_ref[...]
        )
        h_scr[pl.ds(i * _BLK, _BLK), :] = h
        rows = i * _BLK + lax.broadcasted_iota(jnp.int32, (_BLK, 1), 0)
        hm = jnp.where(rows < N, h, 0.0)

        @pl.when(i == 0)
        def _():
            ssum[...] = jnp.zeros_like(ssum)
            ssq[...] = jnp.zeros_like(ssq)

        ssum[...] += jnp.sum(hm, axis=0, keepdims=True)
        ssq[...] += jnp.sum(hm * hm, axis=0, keepdims=True)

    @pl.when(i >= _NBLK)
    def _():
        j = i - _NBLK
        mu = ssum[...] * (1.0 / N)
        var = ssq[...] * (1.0 / N) - mu * mu
        alpha = g1_ref[...] * lax.rsqrt(var + 1e-5)
        c0 = be1_ref[...] - mu * alpha
        h = h_scr[pl.ds(j * _BLK, _BLK), :]
        hn = jnp.maximum(h * alpha + c0, 0.0)
        hw = jnp.dot(hn, w2_ref[...], preferred_element_type=jnp.float32)
        hws = hw * dinv_ref[...]
        hwsa_ref[...] = hws
        hwsb_ref[...] = hws


def _layers_tc(pa, pb, dinv, W1, b1, W2, g1, be1):
    def _ph1(i):
        return (jnp.minimum(i, _NBLK - 1), 0)

    def _ph2(i):
        return (jnp.maximum(i - _NBLK, 0), 0)

    def _mod(i):
        return (lax.rem(i, _NBLK), 0)

    return pl.pallas_call(
        _layers_body,
        grid=(2 * _NBLK,),
        in_specs=[
            pl.BlockSpec((_BLK, OUT), _ph1),
            pl.BlockSpec((_BLK, OUT), _ph1),
            pl.BlockSpec((_BLK, 1), _mod),
            pl.BlockSpec((IN_C, HID), lambda i: (0, 0)),
            pl.BlockSpec((1, HID), lambda i: (0, 0)),
            pl.BlockSpec((HID, OUT), lambda i: (0, 0)),
            pl.BlockSpec((1, HID), lambda i: (0, 0)),
            pl.BlockSpec((1, HID), lambda i: (0, 0)),
        ],
        out_specs=[
            pl.BlockSpec((_BLK, OUT), _ph2),
            pl.BlockSpec((_BLK, OUT), _ph2),
        ],
        out_shape=[
            jax.ShapeDtypeStruct((NPAD, OUT), jnp.float32),
            jax.ShapeDtypeStruct((NPAD, OUT), jnp.float32),
        ],
        scratch_shapes=[
            pltpu.VMEM((NPAD, HID), jnp.float32),
            pltpu.VMEM((1, HID), jnp.float32),
            pltpu.VMEM((1, HID), jnp.float32),
        ],
    )(pa, pb, dinv, W1, b1, W2, g1, be1)


def _final_body(p20_ref, p21_ref, hws_ref, dinv_ref, batch_ref, g2_ref, be2_ref,
                out_ref, ssum, ssq, pooled, cntf):
    i = pl.program_id(0)
    a = (p20_ref[...] + p21_ref[...] - hws_ref[...]) * dinv_ref[...]
    rows = i * _BLK + lax.broadcasted_iota(jnp.int32, (_BLK, 1), 0)
    mask = rows < N
    am = jnp.where(mask, a, 0.0)

    @pl.when(i == 0)
    def _():
        ssum[...] = jnp.zeros_like(ssum)
        ssq[...] = jnp.zeros_like(ssq)
        pooled[...] = jnp.zeros_like(pooled)
        cntf[...] = jnp.zeros_like(cntf)

    ssum[...] += jnp.sum(am, axis=0, keepdims=True)
    ssq[...] += jnp.sum(am * am, axis=0, keepdims=True)
    gids = lax.broadcasted_iota(jnp.int32, (_BLK, NG), 1)
    onehot = jnp.where((batch_ref[...] == gids) & mask, 1.0, 0.0)
    pooled[...] += lax.dot_general(onehot, am, (((0,), (0,)), ((), ())),
                                   preferred_element_type=jnp.float32)
    maskb = jnp.where(mask, 1.0, 0.0) * jnp.ones((_BLK, OUT), jnp.float32)
    cntf[...] += lax.dot_general(onehot, maskb, (((0,), (0,)), ((), ())),
                                 preferred_element_type=jnp.float32)

    @pl.when(i == _NBLK - 1)
    def _():
        mu = ssum[...] * (1.0 / N)
        var = ssq[...] * (1.0 / N) - mu * mu
        inv = lax.rsqrt(var + 1e-5)
        cnt = cntf[...]
        pm = pooled[...] / jnp.maximum(cnt, 1.0)
        res = (pm - mu) * inv * g2_ref[...] + be2_ref[...]
        out_ref[...] = jnp.where(cnt > 0.0, res, 0.0)


def _final_tc(p20, p21, hws, dinv, batchp, g2, be2):
    return pl.pallas_call(
        _final_body,
        grid=(_NBLK,),
        in_specs=[
            pl.BlockSpec((_BLK, OUT), lambda i: (i, 0)),
            pl.BlockSpec((_BLK, OUT), lambda i: (i, 0)),
            pl.BlockSpec((_BLK, OUT), lambda i: (i, 0)),
            pl.BlockSpec((_BLK, 1), lambda i: (i, 0)),
            pl.BlockSpec((_BLK, 1), lambda i: (i, 0)),
            pl.BlockSpec((1, OUT), lambda i: (0, 0)),
            pl.BlockSpec((1, OUT), lambda i: (0, 0)),
        ],
        out_specs=pl.BlockSpec((NG, OUT), lambda i: (0, 0)),
        out_shape=jax.ShapeDtypeStruct((NG, OUT), jnp.float32),
        scratch_shapes=[
            pltpu.VMEM((1, OUT), jnp.float32),
            pltpu.VMEM((1, OUT), jnp.float32),
            pltpu.VMEM((NG, OUT), jnp.float32),
            pltpu.VMEM((NG, OUT), jnp.float32),
        ],
    )(p20, p21, hws, dinv, batchp, g2, be2)


def kernel(x, edge_index, batch, W1, b1, gamma1, beta1, W2, b2, gamma2, beta2):
    del b2
    src = edge_index[0]
    dst = edge_index[1]
    pad_e = DUMP + jnp.arange(EPAD - E, dtype=jnp.int32) % (NPAD - N)
    srcp = jnp.concatenate([src, pad_e]).reshape(ECH, CH)
    dstp = jnp.concatenate([dst, pad_e]).reshape(ECH, CH)
    x_pad = jnp.pad(x, ((0, NPAD - N), (0, 0)))
    batchp = jnp.pad(batch, (0, NPAD - N)).reshape(NPAD, 1)

    degp = _deg_sc(dstp)
    deg0 = degp[0].reshape(NPAD, 1)
    deg1 = degp[1].reshape(NPAD, 1)
    xsa, xsb, dinv = _prep_tc(deg0, deg1, x_pad)
    p1 = _agg1_sc(srcp, dstp, xsa, xsb)
    hwsa, hwsb = _layers_tc(p1[0], p1[1], dinv, W1, b1.reshape(1, HID),
                            W2, gamma1.reshape(1, HID), beta1.reshape(1, HID))
    p2 = _agg2_sc(srcp, dstp, hwsa, hwsb)
    out = _final_tc(p2[0], p2[1], hwsa, dinv, batchp,
                    gamma2.reshape(1, OUT), beta2.reshape(1, OUT))
    return out
